# Initial kernel scaffold; baseline (speedup 1.0000x reference)
import jax
import jax.numpy as jnp
from jax.experimental import pallas as pl


def kernel(x, edge_index, edge_attr, W_in, b_in, W_ep, b_ep, W_l1, b_l1, W_r1, b_r1, W_e1, att1, bias1, W_l2, b_l2, W_r2, b_r2, W_e2, att2, bias2):
    raise NotImplementedError("write your pallas kernel here")


# speedup vs baseline: 5.4359x; 5.4359x over previous
import functools

import jax
import jax.numpy as jnp
from jax import lax
from jax.experimental import pallas as pl
from jax.experimental.pallas import tpu as pltpu

N_NODES = 10000
N_EDGES = 320000


def _affine_body(x_ref, w_ref, b_ref, o_ref, *, act):
    y = jnp.dot(x_ref[...], w_ref[...], preferred_element_type=jnp.float32)
    y = y + b_ref[...]
    if act == "elu":
        y = jnp.where(y > 0, y, jnp.exp(y) - 1.0)
    o_ref[...] = y


def _affine(x, w, b, act=None, block_m=2048):
    m, k = x.shape
    n = w.shape[1]
    grid = (pl.cdiv(m, block_m),)
    return pl.pallas_call(
        functools.partial(_affine_body, act=act),
        grid=grid,
        in_specs=[
            pl.BlockSpec((block_m, k), lambda i: (i, 0)),
            pl.BlockSpec((k, n), lambda i: (0, 0)),
            pl.BlockSpec((1, n), lambda i: (0, 0)),
        ],
        out_specs=pl.BlockSpec((block_m, n), lambda i: (i, 0)),
        out_shape=jax.ShapeDtypeStruct((m, n), jnp.float32),
    )(x, w, b.reshape(1, n))


def _gat_layer(xl, xr, eproj, src2, dst2, att, bias, H, F):
    m = xl[src2] + xr[dst2] + eproj
    m = jnp.where(m > 0, m, 0.2 * m)
    logit = (m.reshape(-1, H, F) * att[None, :, :]).sum(-1)
    alpha = jnp.exp(logit)
    denom = jax.ops.segment_sum(alpha, dst2, num_segments=N_NODES)
    acc = jax.ops.segment_sum(
        (alpha[:, :, None] * xl[src2].reshape(-1, H, F)).reshape(-1, H * F),
        dst2, num_segments=N_NODES)
    out = acc.reshape(N_NODES, H, F) / (denom[:, :, None] + 1e-16)
    return out.reshape(N_NODES, H * F) + bias


def kernel(x, edge_index, edge_attr, W_in, b_in, W_ep, b_ep, W_l1, b_l1,
           W_r1, b_r1, W_e1, att1, bias1, W_l2, b_l2, W_r2, b_r2, W_e2,
           att2, bias2):
    src = edge_index[0]
    dst = edge_index[1]
    loops = jnp.arange(N_NODES, dtype=src.dtype)
    src2 = jnp.concatenate([src, loops])
    dst2 = jnp.concatenate([dst, loops])

    h = _affine(x, W_in, b_in, act="elu")
    ea = _affine(edge_attr, W_ep, b_ep, act="elu")

    ones = jnp.ones((N_EDGES,), jnp.float32)
    deg = jax.ops.segment_sum(ones, dst, num_segments=N_NODES)
    loop_attr = jax.ops.segment_sum(ea, dst, num_segments=N_NODES) \
        / jnp.maximum(deg, 1.0)[:, None]
    ea2 = jnp.concatenate([ea, loop_attr], axis=0)

    xl1 = _affine(h, W_l1, b_l1)
    xr1 = _affine(h, W_r1, b_r1)
    ep1 = _affine(ea2, W_e1, jnp.zeros((128,), jnp.float32))
    h1 = _gat_layer(xl1, xr1, ep1, src2, dst2, att1, bias1, 4, 32)
    h1 = jnp.where(h1 > 0, h1, jnp.exp(h1) - 1.0)

    xl2 = _affine(h1, W_l2, b_l2)
    xr2 = _affine(h1, W_r2, b_r2)
    ep2 = _affine(ea2, W_e2, jnp.zeros((128,), jnp.float32))
    out = _gat_layer(xl2, xr2, ep2, src2, dst2, att2, bias2, 1, 128)
    return out
